# baseline (speedup 1.0000x reference)
import jax
import jax.numpy as jnp
from jax.experimental import pallas as pl
from jax.experimental.pallas import tpu as pltpu

_B = 16
_H = 16
_S = 2048
_D = 64
_HB = 4
_NBUF = 8


def _chunks():
    out = []
    for b in range(_B):
        for hg in range(_H // _HB):
            out.append((0, b, hg))
            out.append((1, b, hg))
    return out


def _body(pos_ref, kc_ref, vc_ref, kval_ref, vval_ref, kout_ref, vout_ref,
          bufs, in_sems, out_sems):
    ins = (kc_ref, vc_ref)
    outs = (kout_ref, vout_ref)
    vals = (kval_ref, vval_ref)
    chunks = _chunks()

    def in_copy(i):
        cache, b, hg = chunks[i]
        j = i % _NBUF
        return pltpu.make_async_copy(
            ins[cache].at[b, pl.ds(hg * _HB, _HB)], bufs.at[j], in_sems.at[j]
        )

    def out_copy(i):
        cache, b, hg = chunks[i]
        j = i % _NBUF
        return pltpu.make_async_copy(
            bufs.at[j], outs[cache].at[b, pl.ds(hg * _HB, _HB)], out_sems.at[j]
        )

    for j in range(_NBUF):
        in_copy(j).start()
    for i, (cache, b, hg) in enumerate(chunks):
        j = i % _NBUF
        if i >= _NBUF:
            out_copy(i - _NBUF).wait()
        in_copy(i).wait()
        r = pos_ref[b] - 1
        bufs[j, :, pl.ds(r, 1), :] = vals[cache][b, pl.ds(hg * _HB, _HB), :, :]
        out_copy(i).start()
        nxt = i + _NBUF
        if nxt < len(chunks):
            in_copy(nxt).start()
    for i in range(len(chunks) - _NBUF, len(chunks)):
        out_copy(i).wait()


def kernel(k_cache, v_cache, k_val, v_val, input_pos):
    out_shape = jax.ShapeDtypeStruct((_B, _H, _S, _D), jnp.float32)
    hbm_spec = pl.BlockSpec(memory_space=pltpu.MemorySpace.HBM)
    vmem_spec = pl.BlockSpec(memory_space=pltpu.MemorySpace.VMEM)
    grid_spec = pltpu.PrefetchScalarGridSpec(
        num_scalar_prefetch=1,
        grid=(),
        in_specs=[hbm_spec, hbm_spec, vmem_spec, vmem_spec],
        out_specs=[hbm_spec, hbm_spec],
        scratch_shapes=[
            pltpu.VMEM((_NBUF, _HB, _S, _D), jnp.float32),
            pltpu.SemaphoreType.DMA((_NBUF,)),
            pltpu.SemaphoreType.DMA((_NBUF,)),
        ],
    )
    return pl.pallas_call(
        _body,
        grid_spec=grid_spec,
        out_shape=[out_shape, out_shape],
        compiler_params=pltpu.CompilerParams(
            vmem_limit_bytes=100 * 1024 * 1024,
        ),
    )(input_pos, k_cache, v_cache, k_val, v_val)
